# baseline (speedup 1.0000x reference)
import functools

import jax
import jax.numpy as jnp
from jax.experimental import pallas as pl
from jax.experimental.pallas import tpu as pltpu
from jax.experimental.pallas import tpu_sc as plsc

B = 4096
L = 50
D_EMB = 64
N_TYPES = 9
LN_EPS = 1e-3

_NC = 2
_NS = 16
_NW = _NC * _NS
_B_BLK = 256
_G = 128
_MESH = dict(core_axis_name="c", subcore_axis_name="s")


def _sc_content_gather(tab2, tok_idx):
    n_tok = tok_idx.size
    tok_pw = n_tok // _NW
    n_ch = tok_pw // _G
    n_pair = n_ch // 2 * 2
    d2 = tab2.shape[1]
    tok3 = tok_idx.reshape(_NW, n_ch, _G)

    @functools.partial(
        pl.kernel, mesh=plsc.VectorSubcoreMesh(**_MESH),
        out_type=jax.ShapeDtypeStruct((n_tok, d2), jnp.float32),
        scratch_types=[pltpu.VMEM((n_ch, _G), jnp.int32),
                       pltpu.VMEM((_G, d2), jnp.float32),
                       pltpu.VMEM((_G, d2), jnp.float32),
                       pltpu.SemaphoreType.DMA,
                       pltpu.SemaphoreType.DMA])
    def gather_kernel(ctab, cidx, cout, cidx_v, rows_a, rows_b, sem_a, sem_b):
        wid = jax.lax.axis_index("s") * _NC + jax.lax.axis_index("c")
        cbase = wid * tok_pw
        pltpu.sync_copy(cidx.at[wid], cidx_v)

        @pl.loop(0, n_pair, step=2)
        def _(c):
            ca = pltpu.async_copy(ctab.at[cidx_v.at[c]], rows_a, sem_a)
            cb = pltpu.async_copy(ctab.at[cidx_v.at[c + 1]], rows_b, sem_b)
            ca.wait()
            pltpu.sync_copy(rows_a, cout.at[pl.ds(cbase + c * _G, _G)])
            cb.wait()
            pltpu.sync_copy(rows_b, cout.at[pl.ds(cbase + (c + 1) * _G, _G)])

        if n_ch != n_pair:
            c = n_pair
            pltpu.async_copy(ctab.at[cidx_v.at[c]], rows_a, sem_a).wait()
            pltpu.sync_copy(rows_a, cout.at[pl.ds(cbase + c * _G, _G)])

    return gather_kernel(tab2, tok3)


def _sc_user_type_gather(utab2, handle_idx, type_table, type_idx):
    u_pw = B // _NW
    d2 = utab2.shape[1]
    d_t = type_table.shape[1]
    hand2 = handle_idx.reshape(_NW, u_pw)
    typ2 = type_idx.reshape(_NW, u_pw)

    @functools.partial(
        pl.kernel, mesh=plsc.VectorSubcoreMesh(**_MESH),
        out_type=(jax.ShapeDtypeStruct((B, d2), jnp.float32),
                  jax.ShapeDtypeStruct((B, d_t), jnp.float32)),
        scratch_types=[pltpu.VMEM((u_pw,), jnp.int32),
                       pltpu.VMEM((u_pw, d2), jnp.float32),
                       pltpu.VMEM((u_pw, d_t), jnp.float32),
                       pltpu.SemaphoreType.DMA])
    def gather_kernel(utab, uidx, ttab, tidx, uout, tout,
                      sidx_v, urows_v, trows_v, sem):
        wid = jax.lax.axis_index("s") * _NC + jax.lax.axis_index("c")
        ubase = wid * u_pw
        pltpu.sync_copy(uidx.at[wid], sidx_v)
        pltpu.async_copy(utab.at[sidx_v], urows_v, sem).wait()
        pltpu.sync_copy(urows_v, uout.at[pl.ds(ubase, u_pw)])

        pltpu.sync_copy(tidx.at[wid], sidx_v)
        pltpu.async_copy(ttab.at[sidx_v], trows_v, sem).wait()
        pltpu.sync_copy(trows_v, tout.at[pl.ds(ubase, u_pw)])

    return gather_kernel(utab2, hand2, type_table, typ2)


def _tc_mlp_kernel(cont_ref, vis_ref, typeb_ref, w1a_ref, w1c_ref, w2_ref,
                   b2_ref, g_ref, beta_ref, wout_ref, out_ref):
    x = cont_ref[...]
    qm = jnp.mean(x, axis=0)[:, :D_EMB]
    h1 = jnp.dot(qm, w1a_ref[...], preferred_element_type=jnp.float32)
    h1 += jnp.dot(vis_ref[...][:, :D_EMB], w1c_ref[...],
                  preferred_element_type=jnp.float32)
    h1 += typeb_ref[...]
    h1 = jnp.maximum(h1, 0.0)
    h2 = jnp.dot(h1, w2_ref[...], preferred_element_type=jnp.float32)
    h2 += b2_ref[...]
    h2 = jnp.maximum(h2, 0.0)
    mu = jnp.mean(h2, axis=-1, keepdims=True)
    dev = h2 - mu
    var = jnp.mean(dev * dev, axis=-1, keepdims=True)
    hn = dev * jax.lax.rsqrt(var + LN_EPS) * g_ref[...] + beta_ref[...]
    out_ref[...] = jnp.dot(hn, wout_ref[...], preferred_element_type=jnp.float32)


def _tc_mlp_half(cont3, vis, typeb, w1a, w1c, W2, b2, ln_gamma, ln_beta,
                 W_out, half):
    n_out = W_out.shape[1]
    half_b = cont3.shape[1]
    off = half * (half_b // _B_BLK)
    grid = (half_b // _B_BLK,)
    return pl.pallas_call(
        _tc_mlp_kernel,
        grid=grid,
        in_specs=[
            pl.BlockSpec((L, _B_BLK, 2 * D_EMB), lambda i: (0, i, 0)),
            pl.BlockSpec((_B_BLK, 2 * D_EMB), lambda i: (i + off, 0)),
            pl.BlockSpec((_B_BLK, W2.shape[0]), lambda i: (i + off, 0)),
            pl.BlockSpec(w1a.shape, lambda i: (0, 0)),
            pl.BlockSpec(w1c.shape, lambda i: (0, 0)),
            pl.BlockSpec(W2.shape, lambda i: (0, 0)),
            pl.BlockSpec((1, W2.shape[1]), lambda i: (0, 0)),
            pl.BlockSpec((1, W2.shape[1]), lambda i: (0, 0)),
            pl.BlockSpec((1, W2.shape[1]), lambda i: (0, 0)),
            pl.BlockSpec(W_out.shape, lambda i: (0, 0)),
        ],
        out_specs=pl.BlockSpec((_B_BLK, n_out), lambda i: (i, 0)),
        out_shape=jax.ShapeDtypeStruct((half_b, n_out), jnp.float32),
    )(cont3, vis, typeb, w1a, w1c, W2, b2.reshape(1, -1),
      ln_gamma.reshape(1, -1), ln_beta.reshape(1, -1), W_out)


def kernel(content_tokens, user_type_idx, user_handle_idx, content_table,
           user_table, W1, b1, W2, b2, ln_gamma, ln_beta, W_out):
    type_table = W1[D_EMB:D_EMB + N_TYPES] + b1[None, :]
    ctab2 = jnp.concatenate([content_table, content_table], axis=1)
    utab2 = jnp.concatenate([user_table, user_table], axis=1)
    w1a = W1[:D_EMB]
    w1c = W1[D_EMB + N_TYPES:]

    half_b = B // 2
    tok_tm = content_tokens.T
    tok_h = [tok_tm[:, :half_b].reshape(L * half_b),
             tok_tm[:, half_b:].reshape(L * half_b)]

    cont0 = _sc_content_gather(ctab2, tok_h[0])
    vis, typeb = _sc_user_type_gather(utab2, user_handle_idx, type_table,
                                      user_type_idx)
    cont1 = _sc_content_gather(ctab2, tok_h[1])

    args = (vis, typeb, w1a, w1c, W2, b2, ln_gamma, ln_beta, W_out)
    out0 = _tc_mlp_half(cont0.reshape(L, half_b, 2 * D_EMB), *args, half=0)
    out1 = _tc_mlp_half(cont1.reshape(L, half_b, 2 * D_EMB), *args, half=1)
    return jnp.concatenate([out0, out1], axis=0)


# speedup vs baseline: 1.0049x; 1.0049x over previous
import functools

import jax
import jax.numpy as jnp
from jax.experimental import pallas as pl
from jax.experimental.pallas import tpu as pltpu
from jax.experimental.pallas import tpu_sc as plsc

B = 4096
L = 50
D_EMB = 64
N_TYPES = 9
LN_EPS = 1e-3

_NC = 2
_NS = 16
_NW = _NC * _NS
_B_BLK = 256
_G = 128
_MESH = dict(core_axis_name="c", subcore_axis_name="s")


def _sc_content_gather(tab2, tok_idx):
    n_tok = tok_idx.size
    tok_pw = n_tok // _NW
    n_ch = tok_pw // _G
    n_quad = n_ch // 4 * 4
    d2 = tab2.shape[1]
    tok3 = tok_idx.reshape(_NW, n_ch, _G)

    @functools.partial(
        pl.kernel, mesh=plsc.VectorSubcoreMesh(**_MESH),
        out_type=jax.ShapeDtypeStruct((n_tok, d2), jnp.float32),
        scratch_types=[pltpu.VMEM((n_ch, _G), jnp.int32),
                       pltpu.VMEM((_G, d2), jnp.float32),
                       pltpu.VMEM((_G, d2), jnp.float32),
                       pltpu.VMEM((_G, d2), jnp.float32),
                       pltpu.VMEM((_G, d2), jnp.float32),
                       pltpu.SemaphoreType.DMA,
                       pltpu.SemaphoreType.DMA,
                       pltpu.SemaphoreType.DMA,
                       pltpu.SemaphoreType.DMA])
    def gather_kernel(ctab, cidx, cout, cidx_v, rows_a, rows_b, rows_c,
                      rows_d, sem_a, sem_b, sem_c, sem_d):
        wid = jax.lax.axis_index("s") * _NC + jax.lax.axis_index("c")
        cbase = wid * tok_pw
        pltpu.sync_copy(cidx.at[wid], cidx_v)
        bufs = ((rows_a, sem_a), (rows_b, sem_b),
                (rows_c, sem_c), (rows_d, sem_d))

        @pl.loop(0, n_quad, step=4)
        def _(c):
            cps = [pltpu.async_copy(ctab.at[cidx_v.at[c + k]], buf, sem)
                   for k, (buf, sem) in enumerate(bufs)]
            for k, (buf, _) in enumerate(bufs):
                cps[k].wait()
                pltpu.sync_copy(buf, cout.at[pl.ds(cbase + (c + k) * _G, _G)])

        for c in range(n_quad, n_ch):
            buf, sem = bufs[c - n_quad]
            pltpu.async_copy(ctab.at[cidx_v.at[c]], buf, sem).wait()
            pltpu.sync_copy(buf, cout.at[pl.ds(cbase + c * _G, _G)])

    return gather_kernel(tab2, tok3)


def _sc_widen_user(user_table):
    v = user_table.shape[0]
    ch = 1024
    main_pw = 3072
    main_rows = main_pw * _NW

    @functools.partial(
        pl.kernel, mesh=plsc.VectorSubcoreMesh(**_MESH),
        out_type=jax.ShapeDtypeStruct((v, 2 * D_EMB), jnp.float32),
        scratch_types=[pltpu.VMEM((ch, D_EMB), jnp.float32)])
    def widen_kernel(utab, out, buf):
        wid = jax.lax.axis_index("s") * _NC + jax.lax.axis_index("c")

        def dup(base, n):
            pltpu.sync_copy(utab.at[pl.ds(base, n)], buf.at[pl.ds(0, n)])
            pltpu.sync_copy(buf.at[pl.ds(0, n)],
                            out.at[pl.ds(base, n), pl.ds(0, D_EMB)])
            pltpu.sync_copy(buf.at[pl.ds(0, n)],
                            out.at[pl.ds(base, n), pl.ds(D_EMB, D_EMB)])

        @pl.loop(0, 3)
        def _(c):
            dup(wid * main_pw + c * ch, ch)

        @pl.when(wid == 0)
        def _():
            dup(main_rows, 848)

        @pl.when(wid == 1)
        def _():
            dup(main_rows + 848, 848)

        @pl.when(wid == 2)
        def _():
            dup(v - 1, 1)

    return widen_kernel(user_table)


def _sc_user_type_gather(utab2, handle_idx, type_table, type_idx):
    u_pw = B // _NW
    d2 = utab2.shape[1]
    d_t = type_table.shape[1]
    hand2 = handle_idx.reshape(_NW, u_pw)
    typ2 = type_idx.reshape(_NW, u_pw)

    @functools.partial(
        pl.kernel, mesh=plsc.VectorSubcoreMesh(**_MESH),
        out_type=(jax.ShapeDtypeStruct((B, d2), jnp.float32),
                  jax.ShapeDtypeStruct((B, d_t), jnp.float32)),
        scratch_types=[pltpu.VMEM((u_pw,), jnp.int32),
                       pltpu.VMEM((u_pw, d2), jnp.float32),
                       pltpu.VMEM((u_pw, d_t), jnp.float32),
                       pltpu.SemaphoreType.DMA])
    def gather_kernel(utab, uidx, ttab, tidx, uout, tout,
                      sidx_v, urows_v, trows_v, sem):
        wid = jax.lax.axis_index("s") * _NC + jax.lax.axis_index("c")
        ubase = wid * u_pw
        pltpu.sync_copy(uidx.at[wid], sidx_v)
        pltpu.async_copy(utab.at[sidx_v], urows_v, sem).wait()
        pltpu.sync_copy(urows_v, uout.at[pl.ds(ubase, u_pw)])

        pltpu.sync_copy(tidx.at[wid], sidx_v)
        pltpu.async_copy(ttab.at[sidx_v], trows_v, sem).wait()
        pltpu.sync_copy(trows_v, tout.at[pl.ds(ubase, u_pw)])

    return gather_kernel(utab2, hand2, type_table, typ2)


def _tc_mlp_kernel(cont_ref, vis_ref, typeb_ref, w1a_ref, w1c_ref, w2_ref,
                   b2_ref, g_ref, beta_ref, wout_ref, out_ref):
    x = cont_ref[...]
    qm = jnp.mean(x, axis=0)[:, :D_EMB]
    h1 = jnp.dot(qm, w1a_ref[...], preferred_element_type=jnp.float32)
    h1 += jnp.dot(vis_ref[...][:, :D_EMB], w1c_ref[...],
                  preferred_element_type=jnp.float32)
    h1 += typeb_ref[...]
    h1 = jnp.maximum(h1, 0.0)
    h2 = jnp.dot(h1, w2_ref[...], preferred_element_type=jnp.float32)
    h2 += b2_ref[...]
    h2 = jnp.maximum(h2, 0.0)
    mu = jnp.mean(h2, axis=-1, keepdims=True)
    dev = h2 - mu
    var = jnp.mean(dev * dev, axis=-1, keepdims=True)
    hn = dev * jax.lax.rsqrt(var + LN_EPS) * g_ref[...] + beta_ref[...]
    out_ref[...] = jnp.dot(hn, wout_ref[...], preferred_element_type=jnp.float32)


def _tc_mlp_half(cont3, vis, typeb, w1a, w1c, W2, b2, ln_gamma, ln_beta,
                 W_out, half):
    n_out = W_out.shape[1]
    half_b = cont3.shape[1]
    off = half * (half_b // _B_BLK)
    grid = (half_b // _B_BLK,)
    return pl.pallas_call(
        _tc_mlp_kernel,
        grid=grid,
        in_specs=[
            pl.BlockSpec((L, _B_BLK, 2 * D_EMB), lambda i: (0, i, 0)),
            pl.BlockSpec((_B_BLK, 2 * D_EMB), lambda i: (i + off, 0)),
            pl.BlockSpec((_B_BLK, W2.shape[0]), lambda i: (i + off, 0)),
            pl.BlockSpec(w1a.shape, lambda i: (0, 0)),
            pl.BlockSpec(w1c.shape, lambda i: (0, 0)),
            pl.BlockSpec(W2.shape, lambda i: (0, 0)),
            pl.BlockSpec((1, W2.shape[1]), lambda i: (0, 0)),
            pl.BlockSpec((1, W2.shape[1]), lambda i: (0, 0)),
            pl.BlockSpec((1, W2.shape[1]), lambda i: (0, 0)),
            pl.BlockSpec(W_out.shape, lambda i: (0, 0)),
        ],
        out_specs=pl.BlockSpec((_B_BLK, n_out), lambda i: (i, 0)),
        out_shape=jax.ShapeDtypeStruct((half_b, n_out), jnp.float32),
    )(cont3, vis, typeb, w1a, w1c, W2, b2.reshape(1, -1),
      ln_gamma.reshape(1, -1), ln_beta.reshape(1, -1), W_out)


def kernel(content_tokens, user_type_idx, user_handle_idx, content_table,
           user_table, W1, b1, W2, b2, ln_gamma, ln_beta, W_out):
    type_table = W1[D_EMB:D_EMB + N_TYPES] + b1[None, :]
    ctab2 = jnp.concatenate([content_table, content_table], axis=1)
    utab2 = jnp.concatenate([user_table, user_table], axis=1)
    w1a = W1[:D_EMB]
    w1c = W1[D_EMB + N_TYPES:]

    half_b = B // 2
    tok_tm = content_tokens.T
    tok_h = [tok_tm[:, :half_b].reshape(L * half_b),
             tok_tm[:, half_b:].reshape(L * half_b)]

    cont0 = _sc_content_gather(ctab2, tok_h[0])
    vis, typeb = _sc_user_type_gather(utab2, user_handle_idx, type_table,
                                      user_type_idx)
    cont1 = _sc_content_gather(ctab2, tok_h[1])

    args = (vis, typeb, w1a, w1c, W2, b2, ln_gamma, ln_beta, W_out)
    out0 = _tc_mlp_half(cont0.reshape(L, half_b, 2 * D_EMB), *args, half=0)
    out1 = _tc_mlp_half(cont1.reshape(L, half_b, 2 * D_EMB), *args, half=1)
    return jnp.concatenate([out0, out1], axis=0)


# speedup vs baseline: 1.0418x; 1.0368x over previous
import functools

import jax
import jax.numpy as jnp
from jax.experimental import pallas as pl
from jax.experimental.pallas import tpu as pltpu
from jax.experimental.pallas import tpu_sc as plsc

B = 4096
L = 50
D_EMB = 64
N_TYPES = 9
LN_EPS = 1e-3

_NC = 2
_NS = 16
_NW = _NC * _NS
_B_BLK = 256
_G = 128
_WIDEN_BLK = 4096


def _dup_kernel(in_ref, out_ref):
    x = in_ref[...]
    out_ref[:, :D_EMB] = x
    out_ref[:, D_EMB:] = x


def _widen(table):
    v = table.shape[0]
    g = (v + _WIDEN_BLK - 1) // _WIDEN_BLK
    return pl.pallas_call(
        _dup_kernel, grid=(g,),
        in_specs=[pl.BlockSpec((_WIDEN_BLK, D_EMB), lambda i: (i, 0))],
        out_specs=pl.BlockSpec((_WIDEN_BLK, 2 * D_EMB), lambda i: (i, 0)),
        out_shape=jax.ShapeDtypeStruct((v, 2 * D_EMB), jnp.float32),
    )(table)


def _sc_gather_all(content_tab2, tok_idx, user_tab2, handle_idx,
                   type_table, type_idx):
    n_tok = tok_idx.size
    tok_pw = n_tok // _NW
    n_ch = tok_pw // _G
    u_pw = B // _NW
    d2 = content_tab2.shape[1]
    d_t = type_table.shape[1]
    tok3 = tok_idx.reshape(_NW, n_ch, _G)
    hand2 = handle_idx.reshape(_NW, u_pw)
    typ2 = type_idx.reshape(_NW, u_pw)
    mesh = plsc.VectorSubcoreMesh(core_axis_name="c", subcore_axis_name="s")

    @functools.partial(
        pl.kernel, mesh=mesh,
        out_type=(jax.ShapeDtypeStruct((n_tok, d2), jnp.float32),
                  jax.ShapeDtypeStruct((B, d2), jnp.float32),
                  jax.ShapeDtypeStruct((B, d_t), jnp.float32)),
        scratch_types=[pltpu.VMEM((n_ch, _G), jnp.int32),
                       pltpu.VMEM((_G, d2), jnp.float32),
                       pltpu.VMEM((_G, d2), jnp.float32),
                       pltpu.VMEM((_G, d2), jnp.float32),
                       pltpu.VMEM((_G, d2), jnp.float32),
                       pltpu.VMEM((u_pw,), jnp.int32),
                       pltpu.VMEM((u_pw, d2), jnp.float32),
                       pltpu.VMEM((u_pw, d_t), jnp.float32),
                       pltpu.SemaphoreType.DMA,
                       pltpu.SemaphoreType.DMA,
                       pltpu.SemaphoreType.DMA,
                       pltpu.SemaphoreType.DMA])
    def gather_kernel(ctab, cidx, utab, uidx, ttab, tidx, cout, uout, tout,
                      cidx_v, rows_a, rows_b, rows_c, rows_d, sidx_v,
                      urows_v, trows_v, sem_a, sem_b, sem_c, sem_d):
        wid = jax.lax.axis_index("s") * _NC + jax.lax.axis_index("c")
        cbase = wid * tok_pw
        pltpu.sync_copy(cidx.at[wid], cidx_v)
        bufs = ((rows_a, sem_a), (rows_b, sem_b),
                (rows_c, sem_c), (rows_d, sem_d))
        n_quad = n_ch // 4 * 4

        @pl.loop(0, n_quad, step=4)
        def _(c):
            cps = [pltpu.async_copy(ctab.at[cidx_v.at[c + k]], buf, sem)
                   for k, (buf, sem) in enumerate(bufs)]
            for k, (buf, _) in enumerate(bufs):
                cps[k].wait()
                pltpu.sync_copy(buf, cout.at[pl.ds(cbase + (c + k) * _G, _G)])

        for c in range(n_quad, n_ch):
            buf, sem = bufs[c - n_quad]
            pltpu.async_copy(ctab.at[cidx_v.at[c]], buf, sem).wait()
            pltpu.sync_copy(buf, cout.at[pl.ds(cbase + c * _G, _G)])

        ubase = wid * u_pw
        pltpu.sync_copy(uidx.at[wid], sidx_v)
        pltpu.async_copy(utab.at[sidx_v], urows_v, sem_a).wait()
        pltpu.sync_copy(urows_v, uout.at[pl.ds(ubase, u_pw)])

        pltpu.sync_copy(tidx.at[wid], sidx_v)
        pltpu.async_copy(ttab.at[sidx_v], trows_v, sem_a).wait()
        pltpu.sync_copy(trows_v, tout.at[pl.ds(ubase, u_pw)])

    return gather_kernel(content_tab2, tok3, user_tab2, hand2,
                         type_table, typ2)


def _tc_mlp_kernel(cont_ref, vis_ref, typeb_ref, w1a_ref, w1c_ref, w2_ref,
                   b2_ref, g_ref, beta_ref, wout_ref, out_ref):
    x = cont_ref[...]
    qm = jnp.mean(x, axis=0)[:, :D_EMB]
    h1 = jnp.dot(qm, w1a_ref[...], preferred_element_type=jnp.float32)
    h1 += jnp.dot(vis_ref[...][:, :D_EMB], w1c_ref[...],
                  preferred_element_type=jnp.float32)
    h1 += typeb_ref[...]
    h1 = jnp.maximum(h1, 0.0)
    h2 = jnp.dot(h1, w2_ref[...], preferred_element_type=jnp.float32)
    h2 += b2_ref[...]
    h2 = jnp.maximum(h2, 0.0)
    mu = jnp.mean(h2, axis=-1, keepdims=True)
    dev = h2 - mu
    var = jnp.mean(dev * dev, axis=-1, keepdims=True)
    hn = dev * jax.lax.rsqrt(var + LN_EPS) * g_ref[...] + beta_ref[...]
    out_ref[...] = jnp.dot(hn, wout_ref[...], preferred_element_type=jnp.float32)


def kernel(content_tokens, user_type_idx, user_handle_idx, content_table,
           user_table, W1, b1, W2, b2, ln_gamma, ln_beta, W_out):
    n_out = W_out.shape[1]
    type_table = W1[D_EMB:D_EMB + N_TYPES] + b1[None, :]
    ctab2 = jnp.concatenate([content_table, content_table], axis=1)
    utab2 = jnp.concatenate([user_table, user_table], axis=1)
    cont, vis, typeb = _sc_gather_all(
        ctab2, content_tokens.T.reshape(B * L), utab2,
        user_handle_idx, type_table, user_type_idx)

    cont3 = cont.reshape(L, B, 2 * D_EMB)
    w1a = W1[:D_EMB]
    w1c = W1[D_EMB + N_TYPES:]

    grid = (B // _B_BLK,)
    return pl.pallas_call(
        _tc_mlp_kernel,
        grid=grid,
        in_specs=[
            pl.BlockSpec((L, _B_BLK, 2 * D_EMB), lambda i: (0, i, 0)),
            pl.BlockSpec((_B_BLK, 2 * D_EMB), lambda i: (i, 0)),
            pl.BlockSpec((_B_BLK, W1.shape[1]), lambda i: (i, 0)),
            pl.BlockSpec(w1a.shape, lambda i: (0, 0)),
            pl.BlockSpec(w1c.shape, lambda i: (0, 0)),
            pl.BlockSpec(W2.shape, lambda i: (0, 0)),
            pl.BlockSpec((1, W2.shape[1]), lambda i: (0, 0)),
            pl.BlockSpec((1, W2.shape[1]), lambda i: (0, 0)),
            pl.BlockSpec((1, W2.shape[1]), lambda i: (0, 0)),
            pl.BlockSpec(W_out.shape, lambda i: (0, 0)),
        ],
        out_specs=pl.BlockSpec((_B_BLK, n_out), lambda i: (i, 0)),
        out_shape=jax.ShapeDtypeStruct((B, n_out), jnp.float32),
    )(cont3, vis, typeb, w1a, w1c, W2, b2.reshape(1, -1),
      ln_gamma.reshape(1, -1), ln_beta.reshape(1, -1), W_out)


# speedup vs baseline: 1.0441x; 1.0021x over previous
import functools

import jax
import jax.numpy as jnp
from jax.experimental import pallas as pl
from jax.experimental.pallas import tpu as pltpu
from jax.experimental.pallas import tpu_sc as plsc

B = 4096
L = 50
D_EMB = 64
N_TYPES = 9
LN_EPS = 1e-3

_NC = 2
_NS = 16
_NW = _NC * _NS
_B_BLK = 512
_G = 128


def _sc_gather_all(content_tab2, tok_idx, user_tab2, handle_idx,
                   type_table, type_idx):
    n_tok = tok_idx.size
    tok_pw = n_tok // _NW
    n_ch = tok_pw // _G
    u_pw = B // _NW
    d2 = content_tab2.shape[1]
    d_t = type_table.shape[1]
    tok3 = tok_idx.reshape(_NW, n_ch, _G)
    hand2 = handle_idx.reshape(_NW, u_pw)
    typ2 = type_idx.reshape(_NW, u_pw)
    mesh = plsc.VectorSubcoreMesh(core_axis_name="c", subcore_axis_name="s")

    @functools.partial(
        pl.kernel, mesh=mesh,
        out_type=(jax.ShapeDtypeStruct((n_tok, d2), jnp.float32),
                  jax.ShapeDtypeStruct((B, d2), jnp.float32),
                  jax.ShapeDtypeStruct((B, d_t), jnp.float32)),
        scratch_types=[pltpu.VMEM((n_ch, _G), jnp.int32),
                       pltpu.VMEM((_G, d2), jnp.float32),
                       pltpu.VMEM((_G, d2), jnp.float32),
                       pltpu.VMEM((_G, d2), jnp.float32),
                       pltpu.VMEM((_G, d2), jnp.float32),
                       pltpu.VMEM((u_pw,), jnp.int32),
                       pltpu.VMEM((u_pw, d2), jnp.float32),
                       pltpu.VMEM((u_pw, d_t), jnp.float32),
                       pltpu.SemaphoreType.DMA,
                       pltpu.SemaphoreType.DMA,
                       pltpu.SemaphoreType.DMA,
                       pltpu.SemaphoreType.DMA])
    def gather_kernel(ctab, cidx, utab, uidx, ttab, tidx, cout, uout, tout,
                      cidx_v, rows_a, rows_b, rows_c, rows_d, sidx_v,
                      urows_v, trows_v, sem_a, sem_b, sem_c, sem_d):
        wid = jax.lax.axis_index("s") * _NC + jax.lax.axis_index("c")
        cbase = wid * tok_pw
        pltpu.sync_copy(cidx.at[wid], cidx_v)
        bufs = ((rows_a, sem_a), (rows_b, sem_b),
                (rows_c, sem_c), (rows_d, sem_d))
        n_quad = n_ch // 4 * 4

        @pl.loop(0, n_quad, step=4)
        def _(c):
            cps = [pltpu.async_copy(ctab.at[cidx_v.at[c + k]], buf, sem)
                   for k, (buf, sem) in enumerate(bufs)]
            for k, (buf, _) in enumerate(bufs):
                cps[k].wait()
                pltpu.sync_copy(buf, cout.at[pl.ds(cbase + (c + k) * _G, _G)])

        for c in range(n_quad, n_ch):
            buf, sem = bufs[c - n_quad]
            pltpu.async_copy(ctab.at[cidx_v.at[c]], buf, sem).wait()
            pltpu.sync_copy(buf, cout.at[pl.ds(cbase + c * _G, _G)])

        ubase = wid * u_pw
        pltpu.sync_copy(uidx.at[wid], sidx_v)
        pltpu.async_copy(utab.at[sidx_v], urows_v, sem_a).wait()
        pltpu.sync_copy(urows_v, uout.at[pl.ds(ubase, u_pw)])

        pltpu.sync_copy(tidx.at[wid], sidx_v)
        pltpu.async_copy(ttab.at[sidx_v], trows_v, sem_a).wait()
        pltpu.sync_copy(trows_v, tout.at[pl.ds(ubase, u_pw)])

    return gather_kernel(content_tab2, tok3, user_tab2, hand2,
                         type_table, typ2)


def _tc_mlp_kernel(cont_ref, vis_ref, typeb_ref, w1a_ref, w1c_ref, w2_ref,
                   b2_ref, g_ref, beta_ref, wout_ref, out_ref):
    x = cont_ref[...]
    qm = jnp.mean(x, axis=0)[:, :D_EMB]
    h1 = jnp.dot(qm, w1a_ref[...], preferred_element_type=jnp.float32)
    h1 += jnp.dot(vis_ref[...][:, :D_EMB], w1c_ref[...],
                  preferred_element_type=jnp.float32)
    h1 += typeb_ref[...]
    h1 = jnp.maximum(h1, 0.0)
    h2 = jnp.dot(h1, w2_ref[...], preferred_element_type=jnp.float32)
    h2 += b2_ref[...]
    h2 = jnp.maximum(h2, 0.0)
    mu = jnp.mean(h2, axis=-1, keepdims=True)
    dev = h2 - mu
    var = jnp.mean(dev * dev, axis=-1, keepdims=True)
    hn = dev * jax.lax.rsqrt(var + LN_EPS) * g_ref[...] + beta_ref[...]
    out_ref[...] = jnp.dot(hn, wout_ref[...], preferred_element_type=jnp.float32)


def kernel(content_tokens, user_type_idx, user_handle_idx, content_table,
           user_table, W1, b1, W2, b2, ln_gamma, ln_beta, W_out):
    n_out = W_out.shape[1]
    type_table = W1[D_EMB:D_EMB + N_TYPES] + b1[None, :]
    ctab2 = jnp.concatenate([content_table, content_table], axis=1)
    utab2 = jnp.concatenate([user_table, user_table], axis=1)
    cont, vis, typeb = _sc_gather_all(
        ctab2, content_tokens.T.reshape(B * L), utab2,
        user_handle_idx, type_table, user_type_idx)

    cont3 = cont.reshape(L, B, 2 * D_EMB)
    w1a = W1[:D_EMB]
    w1c = W1[D_EMB + N_TYPES:]

    grid = (B // _B_BLK,)
    return pl.pallas_call(
        _tc_mlp_kernel,
        grid=grid,
        in_specs=[
            pl.BlockSpec((L, _B_BLK, 2 * D_EMB), lambda i: (0, i, 0)),
            pl.BlockSpec((_B_BLK, 2 * D_EMB), lambda i: (i, 0)),
            pl.BlockSpec((_B_BLK, W1.shape[1]), lambda i: (i, 0)),
            pl.BlockSpec(w1a.shape, lambda i: (0, 0)),
            pl.BlockSpec(w1c.shape, lambda i: (0, 0)),
            pl.BlockSpec(W2.shape, lambda i: (0, 0)),
            pl.BlockSpec((1, W2.shape[1]), lambda i: (0, 0)),
            pl.BlockSpec((1, W2.shape[1]), lambda i: (0, 0)),
            pl.BlockSpec((1, W2.shape[1]), lambda i: (0, 0)),
            pl.BlockSpec(W_out.shape, lambda i: (0, 0)),
        ],
        out_specs=pl.BlockSpec((_B_BLK, n_out), lambda i: (i, 0)),
        out_shape=jax.ShapeDtypeStruct((B, n_out), jnp.float32),
    )(cont3, vis, typeb, w1a, w1c, W2, b2.reshape(1, -1),
      ln_gamma.reshape(1, -1), ln_beta.reshape(1, -1), W_out)


# speedup vs baseline: 1.0608x; 1.0160x over previous
import functools

import jax
import jax.numpy as jnp
from jax.experimental import pallas as pl
from jax.experimental.pallas import tpu as pltpu
from jax.experimental.pallas import tpu_sc as plsc

B = 4096
L = 50
D_EMB = 64
N_TYPES = 9
LN_EPS = 1e-3

_NC = 2
_NS = 16
_NW = _NC * _NS
_B_BLK = 512
_G = 128


def _sc_gather_all(content_tab2, tok_idx, user_tab2, handle_idx,
                   type_table, type_idx):
    n_tok = tok_idx.size
    tok_pw = n_tok // _NW
    n_ch = tok_pw // _G
    u_pw = B // _NW
    d2 = content_tab2.shape[1]
    d_t = type_table.shape[1]
    tok3 = tok_idx.reshape(_NW, n_ch, _G)
    hand2 = handle_idx.reshape(_NW, u_pw)
    typ2 = type_idx.reshape(_NW, u_pw)
    mesh = plsc.VectorSubcoreMesh(core_axis_name="c", subcore_axis_name="s")

    @functools.partial(
        pl.kernel, mesh=mesh,
        out_type=(jax.ShapeDtypeStruct((n_tok, d2), jnp.float32),
                  jax.ShapeDtypeStruct((B, d2), jnp.float32),
                  jax.ShapeDtypeStruct((B, d_t), jnp.float32)),
        scratch_types=[pltpu.VMEM((n_ch, _G), jnp.int32),
                       pltpu.VMEM((_G, d2), jnp.float32),
                       pltpu.VMEM((_G, d2), jnp.float32),
                       pltpu.VMEM((_G, d2), jnp.float32),
                       pltpu.VMEM((_G, d2), jnp.float32),
                       pltpu.VMEM((u_pw,), jnp.int32),
                       pltpu.VMEM((u_pw,), jnp.int32),
                       pltpu.VMEM((u_pw, d2), jnp.float32),
                       pltpu.VMEM((u_pw, d_t), jnp.float32),
                       pltpu.SemaphoreType.DMA,
                       pltpu.SemaphoreType.DMA,
                       pltpu.SemaphoreType.DMA,
                       pltpu.SemaphoreType.DMA,
                       pltpu.SemaphoreType.DMA,
                       pltpu.SemaphoreType.DMA])
    def gather_kernel(ctab, cidx, utab, uidx, ttab, tidx, cout, uout, tout,
                      cidx_v, rows_a, rows_b, rows_c, rows_d, sidx_v, tidx_v,
                      urows_v, trows_v, sem_a, sem_b, sem_c, sem_d,
                      sem_u, sem_t):
        wid = jax.lax.axis_index("s") * _NC + jax.lax.axis_index("c")
        cbase = wid * tok_pw
        pltpu.sync_copy(uidx.at[wid], sidx_v)
        pltpu.sync_copy(tidx.at[wid], tidx_v)
        cu = pltpu.async_copy(utab.at[sidx_v], urows_v, sem_u)
        ct = pltpu.async_copy(ttab.at[tidx_v], trows_v, sem_t)
        pltpu.sync_copy(cidx.at[wid], cidx_v)
        bufs = ((rows_a, sem_a), (rows_b, sem_b),
                (rows_c, sem_c), (rows_d, sem_d))
        n_quad = n_ch // 4 * 4

        @pl.loop(0, n_quad, step=4)
        def _(c):
            cps = [pltpu.async_copy(ctab.at[cidx_v.at[c + k]], buf, sem)
                   for k, (buf, sem) in enumerate(bufs)]
            for k, (buf, _) in enumerate(bufs):
                cps[k].wait()
                pltpu.sync_copy(buf, cout.at[pl.ds(cbase + (c + k) * _G, _G)])

        for c in range(n_quad, n_ch):
            buf, sem = bufs[c - n_quad]
            pltpu.async_copy(ctab.at[cidx_v.at[c]], buf, sem).wait()
            pltpu.sync_copy(buf, cout.at[pl.ds(cbase + c * _G, _G)])

        ubase = wid * u_pw
        cu.wait()
        pltpu.sync_copy(urows_v, uout.at[pl.ds(ubase, u_pw)])
        ct.wait()
        pltpu.sync_copy(trows_v, tout.at[pl.ds(ubase, u_pw)])

    return gather_kernel(content_tab2, tok3, user_tab2, hand2,
                         type_table, typ2)


def _tc_mlp_kernel(cont_ref, vis_ref, typeb_ref, w1a_ref, w1c_ref, w2_ref,
                   b2_ref, g_ref, beta_ref, wout_ref, out_ref):
    x = cont_ref[...]
    qm = jnp.mean(x, axis=0)[:, :D_EMB]
    h1 = jnp.dot(qm, w1a_ref[...], preferred_element_type=jnp.float32)
    h1 += jnp.dot(vis_ref[...][:, :D_EMB], w1c_ref[...],
                  preferred_element_type=jnp.float32)
    h1 += typeb_ref[...]
    h1 = jnp.maximum(h1, 0.0)
    h2 = jnp.dot(h1, w2_ref[...], preferred_element_type=jnp.float32)
    h2 += b2_ref[...]
    h2 = jnp.maximum(h2, 0.0)
    mu = jnp.mean(h2, axis=-1, keepdims=True)
    dev = h2 - mu
    var = jnp.mean(dev * dev, axis=-1, keepdims=True)
    hn = dev * jax.lax.rsqrt(var + LN_EPS) * g_ref[...] + beta_ref[...]
    out_ref[...] = jnp.dot(hn, wout_ref[...], preferred_element_type=jnp.float32)


def kernel(content_tokens, user_type_idx, user_handle_idx, content_table,
           user_table, W1, b1, W2, b2, ln_gamma, ln_beta, W_out):
    n_out = W_out.shape[1]
    type_table = W1[D_EMB:D_EMB + N_TYPES] + b1[None, :]
    ctab2 = jnp.concatenate([content_table, content_table], axis=1)
    utab2 = jnp.concatenate([user_table, user_table], axis=1)
    cont, vis, typeb = _sc_gather_all(
        ctab2, content_tokens.T.reshape(B * L), utab2,
        user_handle_idx, type_table, user_type_idx)

    cont3 = cont.reshape(L, B, 2 * D_EMB)
    w1a = W1[:D_EMB]
    w1c = W1[D_EMB + N_TYPES:]

    grid = (B // _B_BLK,)
    return pl.pallas_call(
        _tc_mlp_kernel,
        grid=grid,
        in_specs=[
            pl.BlockSpec((L, _B_BLK, 2 * D_EMB), lambda i: (0, i, 0)),
            pl.BlockSpec((_B_BLK, 2 * D_EMB), lambda i: (i, 0)),
            pl.BlockSpec((_B_BLK, W1.shape[1]), lambda i: (i, 0)),
            pl.BlockSpec(w1a.shape, lambda i: (0, 0)),
            pl.BlockSpec(w1c.shape, lambda i: (0, 0)),
            pl.BlockSpec(W2.shape, lambda i: (0, 0)),
            pl.BlockSpec((1, W2.shape[1]), lambda i: (0, 0)),
            pl.BlockSpec((1, W2.shape[1]), lambda i: (0, 0)),
            pl.BlockSpec((1, W2.shape[1]), lambda i: (0, 0)),
            pl.BlockSpec(W_out.shape, lambda i: (0, 0)),
        ],
        out_specs=pl.BlockSpec((_B_BLK, n_out), lambda i: (i, 0)),
        out_shape=jax.ShapeDtypeStruct((B, n_out), jnp.float32),
    )(cont3, vis, typeb, w1a, w1c, W2, b2.reshape(1, -1),
      ln_gamma.reshape(1, -1), ln_beta.reshape(1, -1), W_out)
